# baseline (speedup 1.0000x reference)
import jax
import jax.numpy as jnp
from jax import lax
from jax.experimental import pallas as pl
from jax.experimental.pallas import tpu as pltpu
from jax.experimental.pallas import tpu_sc as plsc

N_ROWS = 10000
N_COLS = 128
NW = 32
CHUNK = 312
TAIL = N_ROWS - NW * CHUNK


def _copy_body(emb_hbm, out_hbm):
    wid = lax.axis_index("s") * 2 + lax.axis_index("c")
    base = wid * CHUNK
    pltpu.sync_copy(emb_hbm.at[pl.ds(base, CHUNK)], out_hbm.at[pl.ds(base, CHUNK)])

    @pl.when(wid == 0)
    def _tail():
        pltpu.sync_copy(
            emb_hbm.at[pl.ds(NW * CHUNK, TAIL)], out_hbm.at[pl.ds(NW * CHUNK, TAIL)]
        )


def kernel(data, edge_index, embeddings):
    f = pl.kernel(
        _copy_body,
        out_type=jax.ShapeDtypeStruct((N_ROWS, N_COLS), jnp.float32),
        mesh=plsc.VectorSubcoreMesh(core_axis_name="c", subcore_axis_name="s"),
    )
    return f(embeddings)


# speedup vs baseline: 7.5681x; 7.5681x over previous
import jax
import jax.numpy as jnp
from jax import lax
from jax.experimental import pallas as pl
from jax.experimental.pallas import tpu as pltpu
from jax.experimental.pallas import tpu_sc as plsc

N_ROWS = 10000
N_COLS = 128
NW = 32
CHUNK = 312
SUB = CHUNK // 3
TAIL = N_ROWS - NW * CHUNK


def _copy_body(emb_hbm, out_hbm, buf, tail_buf, sem_in, sem_out):
    wid = lax.axis_index("s") * 2 + lax.axis_index("c")
    base = wid * CHUNK
    in0 = pltpu.async_copy(
        emb_hbm.at[pl.ds(base, SUB)], buf.at[0], sem_in
    )
    in1 = pltpu.async_copy(
        emb_hbm.at[pl.ds(base + SUB, SUB)], buf.at[1], sem_in
    )
    in2 = pltpu.async_copy(
        emb_hbm.at[pl.ds(base + 2 * SUB, SUB)], buf.at[2], sem_in
    )
    in0.wait()
    out0 = pltpu.async_copy(buf.at[0], out_hbm.at[pl.ds(base, SUB)], sem_out)
    in1.wait()
    out1 = pltpu.async_copy(buf.at[1], out_hbm.at[pl.ds(base + SUB, SUB)], sem_out)
    in2.wait()
    out2 = pltpu.async_copy(buf.at[2], out_hbm.at[pl.ds(base + 2 * SUB, SUB)], sem_out)

    @pl.when(wid == 0)
    def _tail():
        pltpu.sync_copy(emb_hbm.at[pl.ds(NW * CHUNK, TAIL)], tail_buf)
        pltpu.sync_copy(tail_buf, out_hbm.at[pl.ds(NW * CHUNK, TAIL)])

    out0.wait()
    out1.wait()
    out2.wait()


def kernel(data, edge_index, embeddings):
    f = pl.kernel(
        _copy_body,
        out_type=jax.ShapeDtypeStruct((N_ROWS, N_COLS), jnp.float32),
        mesh=plsc.VectorSubcoreMesh(core_axis_name="c", subcore_axis_name="s"),
        scratch_types=[
            pltpu.VMEM((3, SUB, N_COLS), jnp.float32),
            pltpu.VMEM((TAIL, N_COLS), jnp.float32),
            pltpu.SemaphoreType.DMA,
            pltpu.SemaphoreType.DMA,
        ],
    )
    return f(embeddings)


# speedup vs baseline: 20.9729x; 2.7712x over previous
import jax
import jax.numpy as jnp
from jax.experimental import pallas as pl

N_ROWS = 10000
N_COLS = 128
BLOCK = 1000


def _copy_block(x_ref, o_ref):
    o_ref[...] = x_ref[...]


def kernel(data, edge_index, embeddings):
    return pl.pallas_call(
        _copy_block,
        grid=(N_ROWS // BLOCK,),
        in_specs=[pl.BlockSpec((BLOCK, N_COLS), lambda i: (i, 0))],
        out_specs=pl.BlockSpec((BLOCK, N_COLS), lambda i: (i, 0)),
        out_shape=jax.ShapeDtypeStruct((N_ROWS, N_COLS), jnp.float32),
    )(embeddings)
